# baseline (speedup 1.0000x reference)
import functools

import jax
import jax.numpy as jnp
from jax import lax
from jax.experimental import pallas as pl
from jax.experimental.pallas import tpu as pltpu
from jax.experimental.pallas import tpu_sc as plsc

N_NODES = 10000
N_EDGES = 320000
FEATS = 128
CLASSES = 16

NCORE = 2
NSUB = 16
NW = NCORE * NSUB

CH = 80
EPT = N_EDGES // NW
ECH = EPT // CH
NCH = N_NODES // CH
KPT = (NCH + NSUB - 1) // NSUB

KA = 10
KC = 8
PF = 5
KT = 6


def _sc_mesh():
    return plsc.VectorSubcoreMesh(
        core_axis_name="c", subcore_axis_name="s",
        num_cores=NCORE, num_subcores=NSUB)


def _phase_a(src, dst, ones16, zeros16):
    @functools.partial(
        pl.kernel,
        out_type=(jax.ShapeDtypeStruct((NCORE, N_NODES, 16), jnp.float32),
                  jax.ShapeDtypeStruct((NCORE, N_NODES, 16), jnp.float32)),
        mesh=_sc_mesh(),
        compiler_params=pltpu.CompilerParams(use_tc_tiling_on_sc=False),
        scratch_types=[
            pltpu.VMEM((ECH, CH), jnp.int32),
            pltpu.VMEM((ECH, CH), jnp.int32),
            pltpu.VMEM((CH, 16), jnp.float32),
            pltpu.VMEM((CH, 16), jnp.float32),
            pltpu.VMEM_SHARED((N_NODES, 16), jnp.float32),
            pltpu.VMEM_SHARED((N_NODES, 16), jnp.float32),
            pltpu.SemaphoreType.DMA((KA,)),
            pltpu.SemaphoreType.DMA((KA,)),
        ],
    )
    def k(src_h, dst_h, ones_h, zer_h, do_out, di_out,
          src_v, dst_v, ones_v, zb_v, do_sh, di_sh, semo, semi):
        c = lax.axis_index("c")
        s = lax.axis_index("s")
        wid = c * NSUB + s
        pltpu.sync_copy(ones_h, ones_v)
        pltpu.sync_copy(zer_h, zb_v)
        pltpu.sync_copy(src_h.at[pl.ds(wid * ECH, ECH)], src_v)
        pltpu.sync_copy(dst_h.at[pl.ds(wid * ECH, ECH)], dst_v)
        for kk in range(KPT):
            cid = s * KPT + kk

            @pl.when(cid < NCH)
            def _():
                pltpu.sync_copy(zb_v, do_sh.at[pl.ds(cid * CH, CH)])
                pltpu.sync_copy(zb_v, di_sh.at[pl.ds(cid * CH, CH)])

        plsc.subcore_barrier()

        def step(i, carry):
            b = lax.rem(i, KA)

            @pl.when(i >= KA)
            def _():
                pltpu.make_async_copy(zer_h, ones_v, semo.at[b]).wait()
                pltpu.make_async_copy(zer_h, ones_v, semi.at[b]).wait()

            pltpu.async_copy(ones_v, do_sh.at[src_v.at[i]], semo.at[b], add=True)
            pltpu.async_copy(ones_v, di_sh.at[dst_v.at[i]], semi.at[b], add=True)
            return carry

        lax.fori_loop(0, ECH, step, 0)
        for bb in range(KA):
            pltpu.make_async_copy(zer_h, ones_v, semo.at[bb]).wait()
            pltpu.make_async_copy(zer_h, ones_v, semi.at[bb]).wait()
        plsc.subcore_barrier()
        for kk in range(KPT):
            cid = s * KPT + kk

            @pl.when(cid < NCH)
            def _():
                pltpu.sync_copy(do_sh.at[pl.ds(cid * CH, CH)], zb_v)
                pltpu.sync_copy(zb_v, do_out.at[c, pl.ds(cid * CH, CH)])
                pltpu.sync_copy(di_sh.at[pl.ds(cid * CH, CH)], ones_v)
                pltpu.sync_copy(ones_v, di_out.at[c, pl.ds(cid * CH, CH)])

    return k(src.reshape(NW * ECH, CH), dst.reshape(NW * ECH, CH), ones16, zeros16)


def _phase_b(x, W1, do_p, di_p):
    blk = 1000
    grid = N_NODES // blk

    def body(x_ref, w_ref, do_ref, di_ref, z_ref, nim_ref, nom_ref):
        deg_o = do_ref[0] + do_ref[1]
        deg_i = di_ref[0] + di_ref[1]
        no = jnp.where(deg_o > 0, lax.rsqrt(jnp.maximum(deg_o, 1e-12)), 0.0)
        ni = jnp.where(deg_i > 0, lax.rsqrt(jnp.maximum(deg_i, 1e-12)), 0.0)
        xw = jnp.dot(x_ref[...], w_ref[...], preferred_element_type=jnp.float32)
        z_ref[...] = (xw * no[:, 0:1]).astype(jnp.bfloat16)
        nim_ref[...] = ni
        nom_ref[...] = no

    return pl.pallas_call(
        body,
        grid=(grid,),
        in_specs=[
            pl.BlockSpec((blk, FEATS), lambda i: (i, 0)),
            pl.BlockSpec((FEATS, FEATS), lambda i: (0, 0)),
            pl.BlockSpec((NCORE, blk, 16), lambda i: (0, i, 0)),
            pl.BlockSpec((NCORE, blk, 16), lambda i: (0, i, 0)),
        ],
        out_specs=[
            pl.BlockSpec((blk, FEATS), lambda i: (i, 0)),
            pl.BlockSpec((blk, 16), lambda i: (i, 0)),
            pl.BlockSpec((blk, 16), lambda i: (i, 0)),
        ],
        out_shape=[
            jax.ShapeDtypeStruct((N_NODES, FEATS), jnp.bfloat16),
            jax.ShapeDtypeStruct((N_NODES, 16), jnp.float32),
            jax.ShapeDtypeStruct((N_NODES, 16), jnp.float32),
        ],
    )(x, W1, do_p, di_p)


def _phase_c12(z, nim16, src, dst, zeros128, zeros16):
    @functools.partial(
        pl.kernel,
        out_type=(jax.ShapeDtypeStruct((NCORE, N_NODES, FEATS), jnp.bfloat16),
                  jax.ShapeDtypeStruct((NCORE, N_NODES, 16), jnp.float32)),
        mesh=_sc_mesh(),
        compiler_params=pltpu.CompilerParams(use_tc_tiling_on_sc=False),
        scratch_types=[
            pltpu.VMEM((ECH, CH), jnp.int32),
            pltpu.VMEM((ECH, CH), jnp.int32),
            pltpu.VMEM((KC, CH, FEATS), jnp.bfloat16),
            pltpu.VMEM((KT, CH, 16), jnp.float32),
            pltpu.VMEM_SHARED((N_NODES, FEATS), jnp.bfloat16),
            pltpu.VMEM_SHARED((N_NODES, 16), jnp.float32),
            pltpu.VMEM_SHARED((N_NODES, 16), jnp.float32),
            pltpu.SemaphoreType.DMA((KC,)),
            pltpu.SemaphoreType.DMA((KC,)),
            pltpu.SemaphoreType.DMA((KT,)),
            pltpu.SemaphoreType.DMA((KT,)),
        ],
    )
    def k(z_h, nim_h, src_h, dst_h, zer128_h, zer16_h, y_out, t_out,
          src_v, dst_v, rows_v, niv_v, y_sh, t_sh, nim_sh,
          gsem, ssem, g2sem, s2sem):
        c = lax.axis_index("c")
        s = lax.axis_index("s")
        wid = c * NSUB + s
        pltpu.sync_copy(zer128_h, rows_v.at[0])
        for kk in range(KPT):
            cid = s * KPT + kk

            @pl.when(cid < NCH)
            def _():
                pltpu.sync_copy(rows_v.at[0], y_sh.at[pl.ds(cid * CH, CH)])
                pltpu.sync_copy(zer16_h, niv_v.at[0])
                pltpu.sync_copy(niv_v.at[0], t_sh.at[pl.ds(cid * CH, CH)])
                pltpu.sync_copy(nim_h.at[pl.ds(cid * CH, CH)], niv_v.at[0])
                pltpu.sync_copy(niv_v.at[0], nim_sh.at[pl.ds(cid * CH, CH)])

        pltpu.sync_copy(src_h.at[pl.ds(wid * ECH, ECH)], src_v)
        pltpu.sync_copy(dst_h.at[pl.ds(wid * ECH, ECH)], dst_v)
        plsc.subcore_barrier()

        def step_y(i, carry):
            @pl.when(i < ECH)
            def _():
                b = lax.rem(i, KC)

                @pl.when(i >= KC)
                def _():
                    pltpu.make_async_copy(zer128_h, rows_v.at[b],
                                          ssem.at[b]).wait()

                pltpu.async_copy(z_h.at[src_v.at[i]], rows_v.at[b], gsem.at[b])

            @pl.when(i >= PF)
            def _():
                b2 = lax.rem(i - PF, KC)
                pltpu.make_async_copy(zer128_h, rows_v.at[b2],
                                      gsem.at[b2]).wait()
                pltpu.async_copy(rows_v.at[b2], y_sh.at[dst_v.at[i - PF]],
                                 ssem.at[b2], add=True)

            return carry

        lax.fori_loop(0, ECH + PF, step_y, 0)
        for bb in range(KC):
            pltpu.make_async_copy(zer128_h, rows_v.at[bb], ssem.at[bb]).wait()

        def step_t(i, carry):
            @pl.when(i < ECH)
            def _():
                bt = lax.rem(i, KT)

                @pl.when(i >= KT)
                def _():
                    pltpu.make_async_copy(zer16_h, niv_v.at[bt],
                                          s2sem.at[bt]).wait()

                pltpu.async_copy(nim_sh.at[dst_v.at[i]], niv_v.at[bt],
                                 g2sem.at[bt])

            @pl.when(i >= PF)
            def _():
                bt2 = lax.rem(i - PF, KT)
                pltpu.make_async_copy(zer16_h, niv_v.at[bt2],
                                      g2sem.at[bt2]).wait()
                pltpu.async_copy(niv_v.at[bt2], t_sh.at[src_v.at[i - PF]],
                                 s2sem.at[bt2], add=True)

            return carry

        lax.fori_loop(0, ECH + PF, step_t, 0)
        for bb in range(KT):
            pltpu.make_async_copy(zer16_h, niv_v.at[bb], s2sem.at[bb]).wait()
        plsc.subcore_barrier()
        for kk in range(KPT):
            cid = s * KPT + kk

            @pl.when(cid < NCH)
            def _():
                pltpu.sync_copy(y_sh.at[pl.ds(cid * CH, CH)], rows_v.at[0])
                pltpu.sync_copy(rows_v.at[0], y_out.at[c, pl.ds(cid * CH, CH)])
                pltpu.sync_copy(t_sh.at[pl.ds(cid * CH, CH)], niv_v.at[0])
                pltpu.sync_copy(niv_v.at[0], t_out.at[c, pl.ds(cid * CH, CH)])

    return k(z, nim16, src.reshape(NW * ECH, CH), dst.reshape(NW * ECH, CH),
             zeros128, zeros16)


def _phase_d(y_p, t_p, nim16, nom16, b1, W2, b2):
    blk = 1000
    grid = N_NODES // blk

    def body(y_ref, t_ref, nim_ref, nom_ref, b1_ref, w2_ref, b2_ref,
             out_ref, acc):
        i = pl.program_id(0)

        @pl.when(i == 0)
        def _():
            acc[...] = jnp.zeros_like(acc)

        y = y_ref[0].astype(jnp.float32) + y_ref[1].astype(jnp.float32)
        h1 = jnp.maximum(nim_ref[:, 0:1] * y + b1_ref[...], 0.0)
        t = t_ref[0, :, 0:1] + t_ref[1, :, 0:1]
        cw = nom_ref[:, 0:1] * t
        acc[...] += jnp.sum(cw * h1, axis=0, keepdims=True)

        @pl.when(i == grid - 1)
        def _():
            out_ref[...] = jnp.dot(acc[...] * (1.0 / N_NODES), w2_ref[...],
                                   preferred_element_type=jnp.float32) + b2_ref[...]

    return pl.pallas_call(
        body,
        grid=(grid,),
        in_specs=[
            pl.BlockSpec((NCORE, blk, FEATS), lambda i: (0, i, 0)),
            pl.BlockSpec((NCORE, blk, 16), lambda i: (0, i, 0)),
            pl.BlockSpec((blk, 16), lambda i: (i, 0)),
            pl.BlockSpec((blk, 16), lambda i: (i, 0)),
            pl.BlockSpec((1, FEATS), lambda i: (0, 0)),
            pl.BlockSpec((FEATS, CLASSES), lambda i: (0, 0)),
            pl.BlockSpec((1, CLASSES), lambda i: (0, 0)),
        ],
        out_specs=pl.BlockSpec((1, CLASSES), lambda i: (0, 0)),
        out_shape=jax.ShapeDtypeStruct((1, CLASSES), jnp.float32),
        scratch_shapes=[pltpu.VMEM((1, FEATS), jnp.float32)],
    )(y_p, t_p, nim16, nom16, b1, W2, b2)


def kernel(x, edge_index, W1, b1, W2, b2):
    src = edge_index[0].astype(jnp.int32)
    dst = edge_index[1].astype(jnp.int32)
    ones16 = jnp.ones((CH, 16), jnp.float32)
    zeros16 = jnp.zeros((CH, 16), jnp.float32)
    zeros128 = jnp.zeros((CH, FEATS), jnp.bfloat16)

    do_p, di_p = _phase_a(src, dst, ones16, zeros16)
    z, nim16, nom16 = _phase_b(x, W1, do_p, di_p)
    y_p, t_p = _phase_c12(z, nim16, src, dst, zeros128, zeros16)
    out = _phase_d(y_p, t_p, nim16, nom16,
                   b1.reshape(1, FEATS), W2, b2.reshape(1, CLASSES))
    return out.reshape(CLASSES)


# speedup vs baseline: 1.0371x; 1.0371x over previous
import functools

import jax
import jax.numpy as jnp
from jax import lax
from jax.experimental import pallas as pl
from jax.experimental.pallas import tpu as pltpu
from jax.experimental.pallas import tpu_sc as plsc

N_NODES = 10000
N_EDGES = 320000
FEATS = 128
CLASSES = 16

NCORE = 2
NSUB = 16
NW = NCORE * NSUB

CH = 80
EPT = N_EDGES // NW
ECH = EPT // CH
NCH = N_NODES // CH
KPT = (NCH + NSUB - 1) // NSUB

KA = 10
KC = 8
PF = 4
KT = 12


def _sc_mesh():
    return plsc.VectorSubcoreMesh(
        core_axis_name="c", subcore_axis_name="s",
        num_cores=NCORE, num_subcores=NSUB)


def _phase_a(src, dst, ones16, zeros16):
    @functools.partial(
        pl.kernel,
        out_type=(jax.ShapeDtypeStruct((NCORE, N_NODES, 16), jnp.float32),
                  jax.ShapeDtypeStruct((NCORE, N_NODES, 16), jnp.float32)),
        mesh=_sc_mesh(),
        compiler_params=pltpu.CompilerParams(use_tc_tiling_on_sc=False),
        scratch_types=[
            pltpu.VMEM((ECH, CH), jnp.int32),
            pltpu.VMEM((ECH, CH), jnp.int32),
            pltpu.VMEM((CH, 16), jnp.float32),
            pltpu.VMEM((CH, 16), jnp.float32),
            pltpu.VMEM_SHARED((N_NODES, 16), jnp.float32),
            pltpu.VMEM_SHARED((N_NODES, 16), jnp.float32),
            pltpu.SemaphoreType.DMA((KA,)),
            pltpu.SemaphoreType.DMA((KA,)),
        ],
    )
    def k(src_h, dst_h, ones_h, zer_h, do_out, di_out,
          src_v, dst_v, ones_v, zb_v, do_sh, di_sh, semo, semi):
        c = lax.axis_index("c")
        s = lax.axis_index("s")
        wid = c * NSUB + s
        pltpu.sync_copy(ones_h, ones_v)
        pltpu.sync_copy(zer_h, zb_v)
        pltpu.sync_copy(src_h.at[pl.ds(wid * ECH, ECH)], src_v)
        pltpu.sync_copy(dst_h.at[pl.ds(wid * ECH, ECH)], dst_v)
        for kk in range(KPT):
            cid = s * KPT + kk

            @pl.when(cid < NCH)
            def _():
                pltpu.sync_copy(zb_v, do_sh.at[pl.ds(cid * CH, CH)])
                pltpu.sync_copy(zb_v, di_sh.at[pl.ds(cid * CH, CH)])

        plsc.subcore_barrier()

        def step(i, carry):
            b = lax.rem(i, KA)

            @pl.when(i >= KA)
            def _():
                pltpu.make_async_copy(zer_h, ones_v, semo.at[b]).wait()
                pltpu.make_async_copy(zer_h, ones_v, semi.at[b]).wait()

            pltpu.async_copy(ones_v, do_sh.at[src_v.at[i]], semo.at[b], add=True)
            pltpu.async_copy(ones_v, di_sh.at[dst_v.at[i]], semi.at[b], add=True)
            return carry

        lax.fori_loop(0, ECH, step, 0)
        for bb in range(KA):
            pltpu.make_async_copy(zer_h, ones_v, semo.at[bb]).wait()
            pltpu.make_async_copy(zer_h, ones_v, semi.at[bb]).wait()
        plsc.subcore_barrier()
        for kk in range(KPT):
            cid = s * KPT + kk

            @pl.when(cid < NCH)
            def _():
                pltpu.sync_copy(do_sh.at[pl.ds(cid * CH, CH)], zb_v)
                pltpu.sync_copy(zb_v, do_out.at[c, pl.ds(cid * CH, CH)])
                pltpu.sync_copy(di_sh.at[pl.ds(cid * CH, CH)], ones_v)
                pltpu.sync_copy(ones_v, di_out.at[c, pl.ds(cid * CH, CH)])

    return k(src.reshape(NW * ECH, CH), dst.reshape(NW * ECH, CH), ones16, zeros16)


def _phase_b(x, W1, do_p, di_p):
    blk = 1000
    grid = N_NODES // blk

    def body(x_ref, w_ref, do_ref, di_ref, z_ref, nim_ref, nom_ref):
        deg_o = do_ref[0] + do_ref[1]
        deg_i = di_ref[0] + di_ref[1]
        no = jnp.where(deg_o > 0, lax.rsqrt(jnp.maximum(deg_o, 1e-12)), 0.0)
        ni = jnp.where(deg_i > 0, lax.rsqrt(jnp.maximum(deg_i, 1e-12)), 0.0)
        xw = jnp.dot(x_ref[...], w_ref[...], preferred_element_type=jnp.float32)
        z_ref[...] = (xw * no[:, 0:1]).astype(jnp.bfloat16)
        nim_ref[...] = ni
        nom_ref[...] = no

    return pl.pallas_call(
        body,
        grid=(grid,),
        in_specs=[
            pl.BlockSpec((blk, FEATS), lambda i: (i, 0)),
            pl.BlockSpec((FEATS, FEATS), lambda i: (0, 0)),
            pl.BlockSpec((NCORE, blk, 16), lambda i: (0, i, 0)),
            pl.BlockSpec((NCORE, blk, 16), lambda i: (0, i, 0)),
        ],
        out_specs=[
            pl.BlockSpec((blk, FEATS), lambda i: (i, 0)),
            pl.BlockSpec((blk, 16), lambda i: (i, 0)),
            pl.BlockSpec((blk, 16), lambda i: (i, 0)),
        ],
        out_shape=[
            jax.ShapeDtypeStruct((N_NODES, FEATS), jnp.bfloat16),
            jax.ShapeDtypeStruct((N_NODES, 16), jnp.float32),
            jax.ShapeDtypeStruct((N_NODES, 16), jnp.float32),
        ],
    )(x, W1, do_p, di_p)


def _phase_c1(z, src, dst, zeros128):
    @functools.partial(
        pl.kernel,
        out_type=jax.ShapeDtypeStruct((NCORE, N_NODES, FEATS), jnp.bfloat16),
        mesh=_sc_mesh(),
        compiler_params=pltpu.CompilerParams(use_tc_tiling_on_sc=False),
        scratch_types=[
            pltpu.VMEM((ECH, CH), jnp.int32),
            pltpu.VMEM((ECH, CH), jnp.int32),
            pltpu.VMEM((KC, CH, FEATS), jnp.bfloat16),
            pltpu.VMEM_SHARED((N_NODES, FEATS), jnp.bfloat16),
            pltpu.SemaphoreType.DMA((KC,)),
            pltpu.SemaphoreType.DMA((KC,)),
        ],
    )
    def k(z_h, src_h, dst_h, zer128_h, y_out,
          src_v, dst_v, rows_v, y_sh, gsem, ssem):
        c = lax.axis_index("c")
        s = lax.axis_index("s")
        wid = c * NSUB + s
        pltpu.sync_copy(zer128_h, rows_v.at[0])
        for kk in range(KPT):
            cid = s * KPT + kk

            @pl.when(cid < NCH)
            def _():
                pltpu.sync_copy(rows_v.at[0], y_sh.at[pl.ds(cid * CH, CH)])

        pltpu.sync_copy(src_h.at[pl.ds(wid * ECH, ECH)], src_v)
        pltpu.sync_copy(dst_h.at[pl.ds(wid * ECH, ECH)], dst_v)
        plsc.subcore_barrier()

        def step(i, carry):
            @pl.when(i < ECH)
            def _():
                b = lax.rem(i, KC)

                @pl.when(i >= KC)
                def _():
                    pltpu.make_async_copy(zer128_h, rows_v.at[b],
                                          ssem.at[b]).wait()

                pltpu.async_copy(z_h.at[src_v.at[i]], rows_v.at[b], gsem.at[b])

            @pl.when(i >= PF)
            def _():
                b2 = lax.rem(i - PF, KC)
                pltpu.make_async_copy(zer128_h, rows_v.at[b2],
                                      gsem.at[b2]).wait()
                pltpu.async_copy(rows_v.at[b2], y_sh.at[dst_v.at[i - PF]],
                                 ssem.at[b2], add=True)

            return carry

        lax.fori_loop(0, ECH + PF, step, 0)
        for bb in range(KC):
            pltpu.make_async_copy(zer128_h, rows_v.at[bb], ssem.at[bb]).wait()
        plsc.subcore_barrier()
        for kk in range(KPT):
            cid = s * KPT + kk

            @pl.when(cid < NCH)
            def _():
                pltpu.sync_copy(y_sh.at[pl.ds(cid * CH, CH)], rows_v.at[0])
                pltpu.sync_copy(rows_v.at[0], y_out.at[c, pl.ds(cid * CH, CH)])

    return k(z, src.reshape(NW * ECH, CH), dst.reshape(NW * ECH, CH), zeros128)


def _phase_c2(nim16, src, dst, zeros16):
    @functools.partial(
        pl.kernel,
        out_type=jax.ShapeDtypeStruct((NCORE, N_NODES, 16), jnp.float32),
        mesh=_sc_mesh(),
        compiler_params=pltpu.CompilerParams(use_tc_tiling_on_sc=False),
        scratch_types=[
            pltpu.VMEM((ECH, CH), jnp.int32),
            pltpu.VMEM((ECH, CH), jnp.int32),
            pltpu.VMEM((KT, CH, 16), jnp.float32),
            pltpu.VMEM_SHARED((N_NODES, 16), jnp.float32),
            pltpu.VMEM_SHARED((N_NODES, 16), jnp.float32),
            pltpu.SemaphoreType.DMA((KT,)),
            pltpu.SemaphoreType.DMA((KT,)),
        ],
    )
    def k(nim_h, src_h, dst_h, zer16_h, t_out,
          src_v, dst_v, niv_v, t_sh, nim_sh, gsem, ssem):
        c = lax.axis_index("c")
        s = lax.axis_index("s")
        wid = c * NSUB + s
        for kk in range(KPT):
            cid = s * KPT + kk

            @pl.when(cid < NCH)
            def _():
                pltpu.sync_copy(zer16_h, niv_v.at[0])
                pltpu.sync_copy(niv_v.at[0], t_sh.at[pl.ds(cid * CH, CH)])
                pltpu.sync_copy(nim_h.at[pl.ds(cid * CH, CH)], niv_v.at[0])
                pltpu.sync_copy(niv_v.at[0], nim_sh.at[pl.ds(cid * CH, CH)])

        pltpu.sync_copy(src_h.at[pl.ds(wid * ECH, ECH)], src_v)
        pltpu.sync_copy(dst_h.at[pl.ds(wid * ECH, ECH)], dst_v)
        plsc.subcore_barrier()

        def step(i, carry):
            @pl.when(i < ECH)
            def _():
                b = lax.rem(i, KT)

                @pl.when(i >= KT)
                def _():
                    pltpu.make_async_copy(zer16_h, niv_v.at[b],
                                          ssem.at[b]).wait()

                pltpu.async_copy(nim_sh.at[dst_v.at[i]], niv_v.at[b],
                                 gsem.at[b])

            @pl.when(i >= PF)
            def _():
                b2 = lax.rem(i - PF, KT)
                pltpu.make_async_copy(zer16_h, niv_v.at[b2],
                                      gsem.at[b2]).wait()
                pltpu.async_copy(niv_v.at[b2], t_sh.at[src_v.at[i - PF]],
                                 ssem.at[b2], add=True)

            return carry

        lax.fori_loop(0, ECH + PF, step, 0)
        for bb in range(KT):
            pltpu.make_async_copy(zer16_h, niv_v.at[bb], ssem.at[bb]).wait()
        plsc.subcore_barrier()
        for kk in range(KPT):
            cid = s * KPT + kk

            @pl.when(cid < NCH)
            def _():
                pltpu.sync_copy(t_sh.at[pl.ds(cid * CH, CH)], niv_v.at[0])
                pltpu.sync_copy(niv_v.at[0], t_out.at[c, pl.ds(cid * CH, CH)])

    return k(nim16, src.reshape(NW * ECH, CH), dst.reshape(NW * ECH, CH),
             zeros16)


def _phase_d(y_p, t_p, nim16, nom16, b1, W2, b2):
    blk = 1000
    grid = N_NODES // blk

    def body(y_ref, t_ref, nim_ref, nom_ref, b1_ref, w2_ref, b2_ref,
             out_ref, acc):
        i = pl.program_id(0)

        @pl.when(i == 0)
        def _():
            acc[...] = jnp.zeros_like(acc)

        y = y_ref[0].astype(jnp.float32) + y_ref[1].astype(jnp.float32)
        h1 = jnp.maximum(nim_ref[:, 0:1] * y + b1_ref[...], 0.0)
        t = t_ref[0, :, 0:1] + t_ref[1, :, 0:1]
        cw = nom_ref[:, 0:1] * t
        acc[...] += jnp.sum(cw * h1, axis=0, keepdims=True)

        @pl.when(i == grid - 1)
        def _():
            out_ref[...] = jnp.dot(acc[...] * (1.0 / N_NODES), w2_ref[...],
                                   preferred_element_type=jnp.float32) + b2_ref[...]

    return pl.pallas_call(
        body,
        grid=(grid,),
        in_specs=[
            pl.BlockSpec((NCORE, blk, FEATS), lambda i: (0, i, 0)),
            pl.BlockSpec((NCORE, blk, 16), lambda i: (0, i, 0)),
            pl.BlockSpec((blk, 16), lambda i: (i, 0)),
            pl.BlockSpec((blk, 16), lambda i: (i, 0)),
            pl.BlockSpec((1, FEATS), lambda i: (0, 0)),
            pl.BlockSpec((FEATS, CLASSES), lambda i: (0, 0)),
            pl.BlockSpec((1, CLASSES), lambda i: (0, 0)),
        ],
        out_specs=pl.BlockSpec((1, CLASSES), lambda i: (0, 0)),
        out_shape=jax.ShapeDtypeStruct((1, CLASSES), jnp.float32),
        scratch_shapes=[pltpu.VMEM((1, FEATS), jnp.float32)],
    )(y_p, t_p, nim16, nom16, b1, W2, b2)


def kernel(x, edge_index, W1, b1, W2, b2):
    src = edge_index[0].astype(jnp.int32)
    dst = edge_index[1].astype(jnp.int32)
    ones16 = jnp.ones((CH, 16), jnp.float32)
    zeros16 = jnp.zeros((CH, 16), jnp.float32)
    zeros128 = jnp.zeros((CH, FEATS), jnp.bfloat16)

    do_p, di_p = _phase_a(src, dst, ones16, zeros16)
    z, nim16, nom16 = _phase_b(x, W1, do_p, di_p)
    y_p = _phase_c1(z, src, dst, zeros128)
    t_p = _phase_c2(nim16, src, dst, zeros16)
    out = _phase_d(y_p, t_p, nim16, nom16,
                   b1.reshape(1, FEATS), W2, b2.reshape(1, CLASSES))
    return out.reshape(CLASSES)
